# baseline (speedup 1.0000x reference)
import functools

import jax
import jax.numpy as jnp
from jax import lax
from jax.experimental import pallas as pl
from jax.experimental.pallas import tpu as pltpu
from jax.experimental.pallas import tpu_sc as plsc

D = 128
B = 2048
S = 16

NC = 2
NS = 16
LANES = 16
NW = NC * NS

CH = 128
DW = D // 2
GPC = CH // S
NBUF = 2

BIG_CHUNKS = (B * S * S) // NW // CH
SELF_CHUNKS = (B * S) // NW // CH
SEG_CHUNKS = BIG_CHUNKS + SELF_CHUNKS
L2_CHUNKS = (B * S) // NW // CH
TGT_ROWS = B // CH

BIG_OUT = B * S
SEG_OUT = B * S + B


def _sc_gather(x, idx_seg, idx_l2, idx_nodes):
  mesh = plsc.VectorSubcoreMesh(core_axis_name="c", subcore_axis_name="s",
                                num_cores=NC, num_subcores=NS)
  out_type = (
      jax.ShapeDtypeStruct((SEG_OUT // 2, 2 * DW), jnp.int32),
      jax.ShapeDtypeStruct((B * S // 2, 2 * DW), jnp.int32),
      jax.ShapeDtypeStruct((B // 2, 2 * DW), jnp.int32),
  )
  scratch = [
      pltpu.VMEM((SEG_CHUNKS, CH), jnp.int32),
      [pltpu.VMEM((CH, DW), jnp.int32) for _ in range(NBUF)],
      pltpu.VMEM((NBUF * GPC // 2, 2 * DW), jnp.int32),
      pltpu.VMEM((CH // 2, 2 * DW), jnp.int32),
      [pltpu.SemaphoreType.DMA for _ in range(NBUF)],
      [pltpu.SemaphoreType.DMA for _ in range(NBUF)],
  ]

  @functools.partial(
      pl.kernel, out_type=out_type, mesh=mesh, scratch_types=scratch,
      compiler_params=pltpu.CompilerParams(use_tc_tiling_on_sc=False,
                                           needs_layout_passes=False))
  def k(x_h, iseg_h, il2_h, inodes_h,
        o_seg, o_l2, o_tgt,
        idx_v, rows, acc_v, pack_v, sem_g, sem_w):
    wid = lax.axis_index("s") * NC + lax.axis_index("c")

    def gather_start(c, j):
      pltpu.async_copy(x_h.at[idx_v.at[c]], rows[j], sem_g[j])

    def gather_wait(c, j):
      pltpu.make_async_copy(x_h.at[idx_v.at[c]], rows[j], sem_g[j]).wait()

    def out_row_of(c):
      return jnp.where(c < BIG_CHUNKS,
                       wid * (BIG_CHUNKS * GPC) + c * GPC,
                       BIG_OUT + wid * (SELF_CHUNKS * GPC)
                       + (c - BIG_CHUNKS) * GPC)

    def accumulate(j):
      for g in range(GPC):
        r = j * GPC + g
        for d in range(DW // LANES):
          sl = pl.ds(d * LANES, LANES)
          vals = [plsc.bitcast(rows[j][g * S + n, sl], jnp.bfloat16)
                  for n in range(S)]
          while len(vals) > 1:
            vals = [vals[2 * k] + vals[2 * k + 1]
                    for k in range(len(vals) // 2)]
          acc_v[r // 2, pl.ds((r % 2) * DW + d * LANES, LANES)] = (
              plsc.bitcast(vals[0], jnp.int32))

    def repack(j, n_rows):
      def rp_body(r, carry):
        for d in range(DW // LANES):
          sl = pl.ds(d * LANES, LANES)
          pack_v[r, sl] = rows[j][2 * r, sl]
          pack_v[r, pl.ds(DW + d * LANES, LANES)] = rows[j][2 * r + 1, sl]
        return carry
      lax.fori_loop(0, n_rows // 2, rp_body, 0)

    pltpu.sync_copy(iseg_h.at[pl.ds(wid * SEG_CHUNKS, SEG_CHUNKS), :], idx_v)
    gather_start(0, 0)
    gather_start(1, 1)

    def seg_body(i, carry):
      c0 = 2 * i
      gather_wait(c0, 0)
      accumulate(0)

      @pl.when(c0 + 2 < SEG_CHUNKS)
      def _():
        gather_start(c0 + 2, 0)

      gather_wait(c0 + 1, 1)
      accumulate(1)

      @pl.when(c0 + 3 < SEG_CHUNKS)
      def _():
        gather_start(c0 + 3, 1)

      pltpu.sync_copy(acc_v, o_seg.at[pl.ds(out_row_of(c0) // 2, GPC), :])
      return carry
    lax.fori_loop(0, SEG_CHUNKS // 2, seg_body, 0)

    pltpu.sync_copy(il2_h.at[pl.ds(wid * L2_CHUNKS, L2_CHUNKS), :],
                    idx_v.at[pl.ds(0, L2_CHUNKS), :])
    l2_base = wid * (L2_CHUNKS * CH // 2)
    for j in range(NBUF):
      gather_start(j, j)
    for c in range(L2_CHUNKS):
      j = c % NBUF
      gather_wait(c, j)
      repack(j, CH)
      if c + NBUF < L2_CHUNKS:
        gather_start(c + NBUF, j)
      pltpu.sync_copy(pack_v, o_l2.at[pl.ds(l2_base + c * (CH // 2),
                                            CH // 2), :])

    @pl.when(wid < TGT_ROWS)
    def _():
      pltpu.sync_copy(inodes_h.at[pl.ds(wid, 1), :], idx_v.at[pl.ds(0, 1), :])
      pltpu.async_copy(x_h.at[idx_v.at[0]], rows[0], sem_g[0]).wait()
      repack(0, CH)
      pltpu.sync_copy(pack_v, o_tgt.at[pl.ds(wid * (CH // 2), CH // 2), :])

  return k(x, idx_seg, idx_l2, idx_nodes)


def _sage(h_self, h_mean, wa_ref, wb_ref, b_ref):
  t = jnp.dot(h_self, wa_ref[...], preferred_element_type=jnp.float32,
              precision=lax.Precision.HIGHEST)
  t = t + jnp.dot(h_mean, wb_ref[...], preferred_element_type=jnp.float32,
                  precision=lax.Precision.HIGHEST)
  t = jnp.maximum(t + b_ref[...], 0.0)
  n = jnp.sqrt(jnp.sum(t * t, axis=1, keepdims=True))
  return t / jnp.where(n > 0, n, 1.0)


def _tc_sage(h0_l2, o_seg, h0_tgt, W0a, W0b, b0, W1a, W1b, b1):
  BLK = 512
  NBLK = BLK * S

  def body(h_ref, sb_ref, t_ref, ss_ref, w0a, w0b, b0_ref, w1a, w1b, b1_ref,
           o_ref):
    f32 = jnp.float32
    h1n = _sage(h_ref[...].astype(f32), sb_ref[...].astype(f32) * (1.0 / S),
                w0a, w0b, b0_ref)
    h1n_mean = jnp.mean(h1n.reshape(BLK, S, D), axis=1)
    h1s = _sage(t_ref[...].astype(f32), ss_ref[...].astype(f32) * (1.0 / S),
                w0a, w0b, b0_ref)
    o_ref[...] = _sage(h1s, h1n_mean, w1a, w1b, b1_ref)

  return pl.pallas_call(
      body,
      grid=(B // BLK,),
      in_specs=[
          pl.BlockSpec((NBLK, D), lambda i: (i, 0)),
          pl.BlockSpec((NBLK, D), lambda i: (i, 0)),
          pl.BlockSpec((BLK, D), lambda i: (i, 0)),
          pl.BlockSpec((BLK, D), lambda i: (i + BIG_OUT // BLK, 0)),
          pl.BlockSpec((D, D), lambda i: (0, 0)),
          pl.BlockSpec((D, D), lambda i: (0, 0)),
          pl.BlockSpec((1, D), lambda i: (0, 0)),
          pl.BlockSpec((D, D), lambda i: (0, 0)),
          pl.BlockSpec((D, D), lambda i: (0, 0)),
          pl.BlockSpec((1, D), lambda i: (0, 0)),
      ],
      out_specs=pl.BlockSpec((BLK, D), lambda i: (i, 0)),
      out_shape=jax.ShapeDtypeStruct((B, D), jnp.float32),
  )(h0_l2, o_seg, h0_tgt, o_seg, W0a, W0b, b0, W1a, W1b, b1)


def kernel(x, nodes, nbr_l1_self, nbr_l2, nbr_l1_of_l2, W0_w, W0_b, W1_w,
           W1_b):
  big = nbr_l1_of_l2.astype(jnp.int32).reshape(NW, BIG_CHUNKS, CH)
  slf = nbr_l1_self.astype(jnp.int32).reshape(NW, SELF_CHUNKS, CH)
  idx_seg = jnp.concatenate([big, slf], axis=1).reshape(NW * SEG_CHUNKS, CH)
  idx_l2 = nbr_l2.astype(jnp.int32).reshape(B * S // CH, CH)
  idx_nodes = nodes.astype(jnp.int32).reshape(TGT_ROWS, CH)

  xw = lax.bitcast_convert_type(
      x.astype(jnp.bfloat16).reshape(x.shape[0], DW, 2), jnp.int32)
  o_seg_w, h0_l2_w, h0_tgt_w = _sc_gather(xw, idx_seg, idx_l2, idx_nodes)

  def to_bf16(w):
    return lax.bitcast_convert_type(w, jnp.bfloat16).reshape(
        2 * w.shape[0], D)

  o_seg = to_bf16(o_seg_w)
  h0_l2 = to_bf16(h0_l2_w)
  h0_tgt = to_bf16(h0_tgt_w)

  W0a, W0b = W0_w[:D], W0_w[D:]
  W1a, W1b = W1_w[:D], W1_w[D:]
  b0 = W0_b.reshape(1, D)
  b1 = W1_b.reshape(1, D)

  return _tc_sage(h0_l2, o_seg, h0_tgt, W0a, W0b, b0, W1a, W1b, b1)


# speedup vs baseline: 8.5066x; 8.5066x over previous
import functools

import jax
import jax.numpy as jnp
from jax import lax
from jax.experimental import pallas as pl
from jax.experimental.pallas import tpu as pltpu
from jax.experimental.pallas import tpu_sc as plsc

D = 128
B = 2048
S = 16

NC = 2
NS = 16
LANES = 16
NW = NC * NS

CH = 128
GPC = CH // S
NBUF = 2

BIG_CHUNKS = (B * S * S) // NW // CH
SELF_CHUNKS = (B * S) // NW // CH
SEG_CHUNKS = BIG_CHUNKS + SELF_CHUNKS
L2_CHUNKS = (B * S) // NW // CH
TGT_ROWS = B // CH

BIG_OUT = B * S
SEG_OUT = B * S + B


def _sc_gather(x, idx_seg, idx_l2, idx_nodes):
  mesh = plsc.VectorSubcoreMesh(core_axis_name="c", subcore_axis_name="s",
                                num_cores=NC, num_subcores=NS)
  out_type = (
      jax.ShapeDtypeStruct((SEG_OUT, D), jnp.float32),
      jax.ShapeDtypeStruct((B * S, D), jnp.float32),
      jax.ShapeDtypeStruct((B, D), jnp.float32),
  )
  scratch = [
      pltpu.VMEM((SEG_CHUNKS, CH), jnp.int32),
      [pltpu.VMEM((CH, D), jnp.float32) for _ in range(NBUF)],
      pltpu.VMEM((NBUF * GPC, D), jnp.float32),
      [pltpu.SemaphoreType.DMA for _ in range(NBUF)],
      [pltpu.SemaphoreType.DMA for _ in range(NBUF)],
  ]

  @functools.partial(pl.kernel, out_type=out_type, mesh=mesh,
                     scratch_types=scratch)
  def k(x_h, iseg_h, il2_h, inodes_h,
        o_seg, o_l2, o_tgt,
        idx_v, rows, acc_v, sem_g, sem_w):
    wid = lax.axis_index("s") * NC + lax.axis_index("c")

    def gather_start(c, j):
      pltpu.async_copy(x_h.at[idx_v.at[c]], rows[j], sem_g[j])

    def gather_wait(c, j):
      pltpu.make_async_copy(x_h.at[idx_v.at[c]], rows[j], sem_g[j]).wait()

    def out_row_of(c):
      return jnp.where(c < BIG_CHUNKS,
                       wid * (BIG_CHUNKS * GPC) + c * GPC,
                       BIG_OUT + wid * (SELF_CHUNKS * GPC)
                       + (c - BIG_CHUNKS) * GPC)

    def accumulate(j):
      for g in range(GPC):
        for d in range(D // LANES):
          a = rows[j][g * S, pl.ds(d * LANES, LANES)]
          for n in range(1, S):
            a = a + rows[j][g * S + n, pl.ds(d * LANES, LANES)]
          acc_v[j * GPC + g, pl.ds(d * LANES, LANES)] = a

    pltpu.sync_copy(iseg_h.at[pl.ds(wid * SEG_CHUNKS, SEG_CHUNKS), :], idx_v)
    gather_start(0, 0)
    gather_start(1, 1)

    def seg_body(i, carry):
      c0 = 2 * i
      gather_wait(c0, 0)
      accumulate(0)

      @pl.when(c0 + 2 < SEG_CHUNKS)
      def _():
        gather_start(c0 + 2, 0)

      gather_wait(c0 + 1, 1)
      accumulate(1)

      @pl.when(c0 + 3 < SEG_CHUNKS)
      def _():
        gather_start(c0 + 3, 1)

      pltpu.sync_copy(acc_v, o_seg.at[pl.ds(out_row_of(c0), 2 * GPC), :])
      return carry
    lax.fori_loop(0, SEG_CHUNKS // 2, seg_body, 0)

    pltpu.sync_copy(il2_h.at[pl.ds(wid * L2_CHUNKS, L2_CHUNKS), :],
                    idx_v.at[pl.ds(0, L2_CHUNKS), :])
    l2_base = wid * (L2_CHUNKS * CH)
    for j in range(NBUF):
      gather_start(j, j)
    for c in range(L2_CHUNKS):
      j = c % NBUF
      gather_wait(c, j)
      pltpu.async_copy(rows[j], o_l2.at[pl.ds(l2_base + c * CH, CH), :],
                       sem_w[j])
      if c + NBUF < L2_CHUNKS:
        pltpu.make_async_copy(rows[j],
                              o_l2.at[pl.ds(l2_base + c * CH, CH), :],
                              sem_w[j]).wait()
        gather_start(c + NBUF, j)
    for c in range(L2_CHUNKS - NBUF, L2_CHUNKS):
      j = c % NBUF
      pltpu.make_async_copy(rows[j], o_l2.at[pl.ds(l2_base + c * CH, CH), :],
                            sem_w[j]).wait()

    @pl.when(wid < TGT_ROWS)
    def _():
      pltpu.sync_copy(inodes_h.at[pl.ds(wid, 1), :], idx_v.at[pl.ds(0, 1), :])
      pltpu.async_copy(x_h.at[idx_v.at[0]], rows[0], sem_g[0]).wait()
      pltpu.sync_copy(rows[0], o_tgt.at[pl.ds(wid * CH, CH), :])

  return k(x, idx_seg, idx_l2, idx_nodes)


def _sage(h_self, h_mean, wa_ref, wb_ref, b_ref):
  t = jnp.dot(h_self, wa_ref[...], preferred_element_type=jnp.float32,
              precision=lax.Precision.HIGHEST)
  t = t + jnp.dot(h_mean, wb_ref[...], preferred_element_type=jnp.float32,
                  precision=lax.Precision.HIGHEST)
  t = jnp.maximum(t + b_ref[...], 0.0)
  n = jnp.sqrt(jnp.sum(t * t, axis=1, keepdims=True))
  return t / jnp.where(n > 0, n, 1.0)


def _tc_sage(h0_l2, o_seg, h0_tgt, W0a, W0b, b0, W1a, W1b, b1):
  BLK = 512
  NBLK = BLK * S

  def body(h_ref, sb_ref, t_ref, ss_ref, w0a, w0b, b0_ref, w1a, w1b, b1_ref,
           o_ref):
    h1n = _sage(h_ref[...], sb_ref[...] * (1.0 / S), w0a, w0b, b0_ref)
    h1n_mean = jnp.mean(h1n.reshape(BLK, S, D), axis=1)
    h1s = _sage(t_ref[...], ss_ref[...] * (1.0 / S), w0a, w0b, b0_ref)
    o_ref[...] = _sage(h1s, h1n_mean, w1a, w1b, b1_ref)

  return pl.pallas_call(
      body,
      grid=(B // BLK,),
      in_specs=[
          pl.BlockSpec((NBLK, D), lambda i: (i, 0)),
          pl.BlockSpec((NBLK, D), lambda i: (i, 0)),
          pl.BlockSpec((BLK, D), lambda i: (i, 0)),
          pl.BlockSpec((BLK, D), lambda i: (i + BIG_OUT // BLK, 0)),
          pl.BlockSpec((D, D), lambda i: (0, 0)),
          pl.BlockSpec((D, D), lambda i: (0, 0)),
          pl.BlockSpec((1, D), lambda i: (0, 0)),
          pl.BlockSpec((D, D), lambda i: (0, 0)),
          pl.BlockSpec((D, D), lambda i: (0, 0)),
          pl.BlockSpec((1, D), lambda i: (0, 0)),
      ],
      out_specs=pl.BlockSpec((BLK, D), lambda i: (i, 0)),
      out_shape=jax.ShapeDtypeStruct((B, D), jnp.float32),
  )(h0_l2, o_seg, h0_tgt, o_seg, W0a, W0b, b0, W1a, W1b, b1)


def kernel(x, nodes, nbr_l1_self, nbr_l2, nbr_l1_of_l2, W0_w, W0_b, W1_w,
           W1_b):
  big = nbr_l1_of_l2.astype(jnp.int32).reshape(NW, BIG_CHUNKS, CH)
  slf = nbr_l1_self.astype(jnp.int32).reshape(NW, SELF_CHUNKS, CH)
  idx_seg = jnp.concatenate([big, slf], axis=1).reshape(NW * SEG_CHUNKS, CH)
  idx_l2 = nbr_l2.astype(jnp.int32).reshape(B * S // CH, CH)
  idx_nodes = nodes.astype(jnp.int32).reshape(TGT_ROWS, CH)

  o_seg, h0_l2, h0_tgt = _sc_gather(x, idx_seg, idx_l2, idx_nodes)

  W0a, W0b = W0_w[:D], W0_w[D:]
  W1a, W1b = W1_w[:D], W1_w[D:]
  b0 = W0_b.reshape(1, D)
  b1 = W1_b.reshape(1, D)

  return _tc_sage(h0_l2, o_seg, h0_tgt, W0a, W0b, b0, W1a, W1b, b1)


# speedup vs baseline: 9.0476x; 1.0636x over previous
import functools

import jax
import jax.numpy as jnp
from jax import lax
from jax.experimental import pallas as pl
from jax.experimental.pallas import tpu as pltpu
from jax.experimental.pallas import tpu_sc as plsc

D = 128
B = 2048
S = 16

NC = 2
NS = 16
LANES = 16
NW = NC * NS

CH = 128
GPC = CH // S
NBUF = 2

BIG_CHUNKS = (B * S * S) // NW // CH
SELF_CHUNKS = (B * S) // NW // CH
SEG_CHUNKS = BIG_CHUNKS + SELF_CHUNKS
L2_CHUNKS = (B * S) // NW // CH
TGT_ROWS = B // CH

BIG_OUT = B * S
SEG_OUT = B * S + B


def _sc_gather(x, idx_seg, idx_l2, idx_nodes):
  mesh = plsc.VectorSubcoreMesh(core_axis_name="c", subcore_axis_name="s",
                                num_cores=NC, num_subcores=NS)
  out_type = (
      jax.ShapeDtypeStruct((SEG_OUT, D), jnp.float32),
      jax.ShapeDtypeStruct((B * S, D), jnp.float32),
      jax.ShapeDtypeStruct((B, D), jnp.float32),
  )
  scratch = [
      pltpu.VMEM((SEG_CHUNKS, CH), jnp.int32),
      [pltpu.VMEM((CH, D), jnp.float32) for _ in range(NBUF)],
      pltpu.VMEM((NBUF * GPC, D), jnp.float32),
      [pltpu.SemaphoreType.DMA for _ in range(NBUF)],
      [pltpu.SemaphoreType.DMA for _ in range(NBUF)],
  ]

  @functools.partial(pl.kernel, out_type=out_type, mesh=mesh,
                     scratch_types=scratch)
  def k(x_h, iseg_h, il2_h, inodes_h,
        o_seg, o_l2, o_tgt,
        idx_v, rows, acc_v, sem_g, sem_w):
    wid = lax.axis_index("s") * NC + lax.axis_index("c")

    def gather_start(c, j):
      pltpu.async_copy(x_h.at[idx_v.at[c]], rows[j], sem_g[j])

    def gather_wait(c, j):
      pltpu.make_async_copy(x_h.at[idx_v.at[c]], rows[j], sem_g[j]).wait()

    def out_row_of(c):
      return jnp.where(c < BIG_CHUNKS,
                       wid * (BIG_CHUNKS * GPC) + c * GPC,
                       BIG_OUT + wid * (SELF_CHUNKS * GPC)
                       + (c - BIG_CHUNKS) * GPC)

    def accumulate(j):
      for g in range(GPC):
        for d in range(D // LANES):
          a = rows[j][g * S, pl.ds(d * LANES, LANES)]
          for n in range(1, S):
            a = a + rows[j][g * S + n, pl.ds(d * LANES, LANES)]
          acc_v[j * GPC + g, pl.ds(d * LANES, LANES)] = a

    pltpu.sync_copy(iseg_h.at[pl.ds(wid * SEG_CHUNKS, SEG_CHUNKS), :], idx_v)
    gather_start(0, 0)
    gather_start(1, 1)

    def seg_body(i, carry):
      c0 = 2 * i
      gather_wait(c0, 0)
      accumulate(0)

      @pl.when(c0 + 2 < SEG_CHUNKS)
      def _():
        gather_start(c0 + 2, 0)

      gather_wait(c0 + 1, 1)
      accumulate(1)

      @pl.when(c0 + 3 < SEG_CHUNKS)
      def _():
        gather_start(c0 + 3, 1)

      pltpu.sync_copy(acc_v, o_seg.at[pl.ds(out_row_of(c0), 2 * GPC), :])
      return carry
    lax.fori_loop(0, SEG_CHUNKS // 2, seg_body, 0)

    pltpu.sync_copy(il2_h.at[pl.ds(wid * L2_CHUNKS, L2_CHUNKS), :],
                    idx_v.at[pl.ds(0, L2_CHUNKS), :])
    l2_base = wid * (L2_CHUNKS * CH)
    for j in range(NBUF):
      gather_start(j, j)
    for c in range(L2_CHUNKS):
      j = c % NBUF
      gather_wait(c, j)
      pltpu.async_copy(rows[j], o_l2.at[pl.ds(l2_base + c * CH, CH), :],
                       sem_w[j])
      if c + NBUF < L2_CHUNKS:
        pltpu.make_async_copy(rows[j],
                              o_l2.at[pl.ds(l2_base + c * CH, CH), :],
                              sem_w[j]).wait()
        gather_start(c + NBUF, j)
    for c in range(L2_CHUNKS - NBUF, L2_CHUNKS):
      j = c % NBUF
      pltpu.make_async_copy(rows[j], o_l2.at[pl.ds(l2_base + c * CH, CH), :],
                            sem_w[j]).wait()

    @pl.when(wid < TGT_ROWS)
    def _():
      pltpu.sync_copy(inodes_h.at[pl.ds(wid, 1), :], idx_v.at[pl.ds(0, 1), :])
      pltpu.async_copy(x_h.at[idx_v.at[0]], rows[0], sem_g[0]).wait()
      pltpu.sync_copy(rows[0], o_tgt.at[pl.ds(wid * CH, CH), :])

  return k(x, idx_seg, idx_l2, idx_nodes)


def _sage(h_self, h_mean, wa_ref, wb_ref, b_ref):
  t = jnp.dot(h_self, wa_ref[...], preferred_element_type=jnp.float32)
  t = t + jnp.dot(h_mean, wb_ref[...], preferred_element_type=jnp.float32)
  t = jnp.maximum(t + b_ref[...], 0.0)
  n = jnp.sqrt(jnp.sum(t * t, axis=1, keepdims=True))
  return t / jnp.where(n > 0, n, 1.0)


def _tc_sage(h0_l2, o_seg, h0_tgt, W0a, W0b, b0, W1a, W1b, b1):
  BLK = 512
  NBLK = BLK * S

  def body(h_ref, sb_ref, t_ref, ss_ref, w0a, w0b, b0_ref, w1a, w1b, b1_ref,
           o_ref):
    h1n = _sage(h_ref[...], sb_ref[...] * (1.0 / S), w0a, w0b, b0_ref)
    h1n_mean = jnp.mean(h1n.reshape(BLK, S, D), axis=1)
    h1s = _sage(t_ref[...], ss_ref[...] * (1.0 / S), w0a, w0b, b0_ref)
    o_ref[...] = _sage(h1s, h1n_mean, w1a, w1b, b1_ref)

  return pl.pallas_call(
      body,
      grid=(B // BLK,),
      in_specs=[
          pl.BlockSpec((NBLK, D), lambda i: (i, 0)),
          pl.BlockSpec((NBLK, D), lambda i: (i, 0)),
          pl.BlockSpec((BLK, D), lambda i: (i, 0)),
          pl.BlockSpec((BLK, D), lambda i: (i + BIG_OUT // BLK, 0)),
          pl.BlockSpec((D, D), lambda i: (0, 0)),
          pl.BlockSpec((D, D), lambda i: (0, 0)),
          pl.BlockSpec((1, D), lambda i: (0, 0)),
          pl.BlockSpec((D, D), lambda i: (0, 0)),
          pl.BlockSpec((D, D), lambda i: (0, 0)),
          pl.BlockSpec((1, D), lambda i: (0, 0)),
      ],
      out_specs=pl.BlockSpec((BLK, D), lambda i: (i, 0)),
      out_shape=jax.ShapeDtypeStruct((B, D), jnp.float32),
  )(h0_l2, o_seg, h0_tgt, o_seg, W0a, W0b, b0, W1a, W1b, b1)


def kernel(x, nodes, nbr_l1_self, nbr_l2, nbr_l1_of_l2, W0_w, W0_b, W1_w,
           W1_b):
  big = nbr_l1_of_l2.astype(jnp.int32).reshape(NW, BIG_CHUNKS, CH)
  slf = nbr_l1_self.astype(jnp.int32).reshape(NW, SELF_CHUNKS, CH)
  idx_seg = jnp.concatenate([big, slf], axis=1).reshape(NW * SEG_CHUNKS, CH)
  idx_l2 = nbr_l2.astype(jnp.int32).reshape(B * S // CH, CH)
  idx_nodes = nodes.astype(jnp.int32).reshape(TGT_ROWS, CH)

  o_seg, h0_l2, h0_tgt = _sc_gather(x, idx_seg, idx_l2, idx_nodes)

  W0a, W0b = W0_w[:D], W0_w[D:]
  W1a, W1b = W1_w[:D], W1_w[D:]
  b0 = W0_b.reshape(1, D)
  b1 = W1_b.reshape(1, D)

  return _tc_sage(h0_l2, o_seg, h0_tgt, W0a, W0b, b0, W1a, W1b, b1)


# speedup vs baseline: 9.1971x; 1.0165x over previous
import functools

import jax
import jax.numpy as jnp
from jax import lax
from jax.experimental import pallas as pl
from jax.experimental.pallas import tpu as pltpu
from jax.experimental.pallas import tpu_sc as plsc

D = 128
B = 2048
S = 16

NC = 2
NS = 16
LANES = 16
NW = NC * NS

CH = 128
GPC = CH // S
NBUF = 2

BIG_CHUNKS = (B * S * S) // NW // CH
SELF_CHUNKS = (B * S) // NW // CH
SEG_CHUNKS = BIG_CHUNKS + SELF_CHUNKS
L2_CHUNKS = (B * S) // NW // CH
TGT_ROWS = B // CH

BIG_OUT = B * S
SEG_OUT = B * S + B


def _sc_gather(x, idx_seg, idx_l2, idx_nodes):
  mesh = plsc.VectorSubcoreMesh(core_axis_name="c", subcore_axis_name="s",
                                num_cores=NC, num_subcores=NS)
  out_type = (
      jax.ShapeDtypeStruct((SEG_OUT, D), jnp.float32),
      jax.ShapeDtypeStruct((B * S, D), jnp.float32),
      jax.ShapeDtypeStruct((B, D), jnp.float32),
  )
  scratch = [
      pltpu.VMEM((SEG_CHUNKS, CH), jnp.int32),
      [pltpu.VMEM((CH, D), jnp.float32) for _ in range(NBUF)],
      pltpu.VMEM((NBUF * GPC, D), jnp.float32),
      [pltpu.SemaphoreType.DMA for _ in range(NBUF)],
      [pltpu.SemaphoreType.DMA for _ in range(NBUF)],
  ]

  @functools.partial(pl.kernel, out_type=out_type, mesh=mesh,
                     scratch_types=scratch)
  def k(x_h, iseg_h, il2_h, inodes_h,
        o_seg, o_l2, o_tgt,
        idx_v, rows, acc_v, sem_g, sem_w):
    wid = lax.axis_index("s") * NC + lax.axis_index("c")

    def gather_start(c, j):
      pltpu.async_copy(x_h.at[idx_v.at[c]], rows[j], sem_g[j])

    def gather_wait(c, j):
      pltpu.make_async_copy(x_h.at[idx_v.at[c]], rows[j], sem_g[j]).wait()

    def out_row_of(c):
      return jnp.where(c < BIG_CHUNKS,
                       wid * (BIG_CHUNKS * GPC) + c * GPC,
                       BIG_OUT + wid * (SELF_CHUNKS * GPC)
                       + (c - BIG_CHUNKS) * GPC)

    def accumulate(j):
      for g in range(GPC):
        for d in range(D // LANES):
          a = rows[j][g * S, pl.ds(d * LANES, LANES)]
          for n in range(1, S):
            a = a + rows[j][g * S + n, pl.ds(d * LANES, LANES)]
          acc_v[j * GPC + g, pl.ds(d * LANES, LANES)] = a

    pltpu.sync_copy(iseg_h.at[pl.ds(wid * SEG_CHUNKS, SEG_CHUNKS), :], idx_v)
    gather_start(0, 0)
    gather_start(1, 1)

    def seg_write_wait(c0):
      pltpu.make_async_copy(acc_v, o_seg.at[pl.ds(out_row_of(c0), 2 * GPC), :],
                            sem_w[0]).wait()

    def seg_body(i, carry):
      c0 = 2 * i
      gather_wait(c0, 0)

      @pl.when(i > 0)
      def _():
        seg_write_wait(c0 - 2)

      accumulate(0)

      @pl.when(c0 + 2 < SEG_CHUNKS)
      def _():
        gather_start(c0 + 2, 0)

      gather_wait(c0 + 1, 1)
      accumulate(1)

      @pl.when(c0 + 3 < SEG_CHUNKS)
      def _():
        gather_start(c0 + 3, 1)

      pltpu.async_copy(acc_v, o_seg.at[pl.ds(out_row_of(c0), 2 * GPC), :],
                       sem_w[0])
      return carry
    lax.fori_loop(0, SEG_CHUNKS // 2, seg_body, 0)
    seg_write_wait(SEG_CHUNKS - 2)

    pltpu.sync_copy(il2_h.at[pl.ds(wid * L2_CHUNKS, L2_CHUNKS), :],
                    idx_v.at[pl.ds(0, L2_CHUNKS), :])
    l2_base = wid * (L2_CHUNKS * CH)
    for j in range(NBUF):
      gather_start(j, j)
    for c in range(L2_CHUNKS):
      j = c % NBUF
      gather_wait(c, j)
      pltpu.async_copy(rows[j], o_l2.at[pl.ds(l2_base + c * CH, CH), :],
                       sem_w[j])
      if c + NBUF < L2_CHUNKS:
        pltpu.make_async_copy(rows[j],
                              o_l2.at[pl.ds(l2_base + c * CH, CH), :],
                              sem_w[j]).wait()
        gather_start(c + NBUF, j)
    for c in range(L2_CHUNKS - NBUF, L2_CHUNKS):
      j = c % NBUF
      pltpu.make_async_copy(rows[j], o_l2.at[pl.ds(l2_base + c * CH, CH), :],
                            sem_w[j]).wait()

    @pl.when(wid < TGT_ROWS)
    def _():
      pltpu.sync_copy(inodes_h.at[pl.ds(wid, 1), :], idx_v.at[pl.ds(0, 1), :])
      pltpu.async_copy(x_h.at[idx_v.at[0]], rows[0], sem_g[0]).wait()
      pltpu.sync_copy(rows[0], o_tgt.at[pl.ds(wid * CH, CH), :])

  return k(x, idx_seg, idx_l2, idx_nodes)


def _sage(h_self, h_mean, wa_ref, wb_ref, b_ref):
  t = jnp.dot(h_self, wa_ref[...], preferred_element_type=jnp.float32)
  t = t + jnp.dot(h_mean, wb_ref[...], preferred_element_type=jnp.float32)
  t = jnp.maximum(t + b_ref[...], 0.0)
  n = jnp.sqrt(jnp.sum(t * t, axis=1, keepdims=True))
  return t / jnp.where(n > 0, n, 1.0)


def _tc_sage(h0_l2, o_seg, h0_tgt, W0a, W0b, b0, W1a, W1b, b1):
  BLK = 512
  NBLK = BLK * S

  def body(h_ref, sb_ref, t_ref, ss_ref, w0a, w0b, b0_ref, w1a, w1b, b1_ref,
           o_ref):
    h1n = _sage(h_ref[...], sb_ref[...] * (1.0 / S), w0a, w0b, b0_ref)
    h1n_mean = jnp.mean(h1n.reshape(BLK, S, D), axis=1)
    h1s = _sage(t_ref[...], ss_ref[...] * (1.0 / S), w0a, w0b, b0_ref)
    o_ref[...] = _sage(h1s, h1n_mean, w1a, w1b, b1_ref)

  return pl.pallas_call(
      body,
      grid=(B // BLK,),
      in_specs=[
          pl.BlockSpec((NBLK, D), lambda i: (i, 0)),
          pl.BlockSpec((NBLK, D), lambda i: (i, 0)),
          pl.BlockSpec((BLK, D), lambda i: (i, 0)),
          pl.BlockSpec((BLK, D), lambda i: (i + BIG_OUT // BLK, 0)),
          pl.BlockSpec((D, D), lambda i: (0, 0)),
          pl.BlockSpec((D, D), lambda i: (0, 0)),
          pl.BlockSpec((1, D), lambda i: (0, 0)),
          pl.BlockSpec((D, D), lambda i: (0, 0)),
          pl.BlockSpec((D, D), lambda i: (0, 0)),
          pl.BlockSpec((1, D), lambda i: (0, 0)),
      ],
      out_specs=pl.BlockSpec((BLK, D), lambda i: (i, 0)),
      out_shape=jax.ShapeDtypeStruct((B, D), jnp.float32),
  )(h0_l2, o_seg, h0_tgt, o_seg, W0a, W0b, b0, W1a, W1b, b1)


def kernel(x, nodes, nbr_l1_self, nbr_l2, nbr_l1_of_l2, W0_w, W0_b, W1_w,
           W1_b):
  big = nbr_l1_of_l2.astype(jnp.int32).reshape(NW, BIG_CHUNKS, CH)
  slf = nbr_l1_self.astype(jnp.int32).reshape(NW, SELF_CHUNKS, CH)
  idx_seg = jnp.concatenate([big, slf], axis=1).reshape(NW * SEG_CHUNKS, CH)
  idx_l2 = nbr_l2.astype(jnp.int32).reshape(B * S // CH, CH)
  idx_nodes = nodes.astype(jnp.int32).reshape(TGT_ROWS, CH)

  o_seg, h0_l2, h0_tgt = _sc_gather(x, idx_seg, idx_l2, idx_nodes)

  W0a, W0b = W0_w[:D], W0_w[D:]
  W1a, W1b = W1_w[:D], W1_w[D:]
  b0 = W0_b.reshape(1, D)
  b1 = W1_b.reshape(1, D)

  return _tc_sage(h0_l2, o_seg, h0_tgt, W0a, W0b, b0, W1a, W1b, b1)
